# Initial kernel scaffold; baseline (speedup 1.0000x reference)
import jax
import jax.numpy as jnp
from jax.experimental import pallas as pl


def kernel(x, edge_index, edge_attr, batch, node_emb, edge_emb, edge_enc_W, edge_enc_b, pre_W, pre_b, post_W, post_b, lin_W, lin_b, bn_w, bn_b, mlp_W1, mlp_b1, mlp_W2, mlp_b2, mlp_W3, mlp_b3):
    raise NotImplementedError("write your pallas kernel here")


# speedup vs baseline: 25.1173x; 25.1173x over previous
import functools
import jax
import jax.numpy as jnp
from jax import lax
from jax.experimental import pallas as pl
from jax.experimental.pallas import tpu as pltpu
from jax.experimental.pallas import tpu_sc as plsc

N = 10000
NP = 10240
E = 160000
B = 128
L = 4
T = 5
F = 75
TF = 384
AVG_DEG_LOG = 1.0
NODES_PER_W = 320
CELL = 64
CH = 16
BIG = 1e30


def _tc_pre(h, Wsf, Wdf, pbf):
    R = 1024
    grid = (NP // R,)

    def body(h_ref, ws_ref, wd_ref, pb_ref, u_ref, a_ref):
        hb = h_ref[...]
        u_ref[...] = jnp.dot(hb, ws_ref[...], preferred_element_type=jnp.float32)
        a_ref[...] = jnp.dot(hb, wd_ref[...], preferred_element_type=jnp.float32) + pb_ref[0:1, :]

    return pl.pallas_call(
        body,
        grid=grid,
        in_specs=[
            pl.BlockSpec((R, 128), lambda i: (i, 0)),
            pl.BlockSpec((128, TF), lambda i: (0, 0)),
            pl.BlockSpec((128, TF), lambda i: (0, 0)),
            pl.BlockSpec((8, TF), lambda i: (0, 0)),
        ],
        out_specs=[
            pl.BlockSpec((R, TF), lambda i: (i, 0)),
            pl.BlockSpec((R, TF), lambda i: (i, 0)),
        ],
        out_shape=[
            jax.ShapeDtypeStruct((NP, TF), jnp.float32),
            jax.ShapeDtypeStruct((NP, TF), jnp.float32),
        ],
    )(h, Wsf, Wdf, pbf)


def _tc_post(h, a, S, Q, MN, MX, degb, Qx, Q1, Q2, Q3, qbf, lWp, lbf):
    R = 1024
    grid = (NP // R,)

    def body(h_ref, a_ref, s_ref, q_ref, mn_ref, mx_ref, d_ref,
             qx_ref, q1_ref, q2_ref, q3_ref, qb_ref, lw_ref, lb_ref, y_ref):
        deg = d_ref[...]
        av = a_ref[...]
        Sv = s_ref[...]
        degc = jnp.maximum(deg, 1.0)
        logd = jnp.log(degc + 1.0)
        s1 = logd / AVG_DEG_LOG
        s2 = AVG_DEG_LOG / logd
        mean = (deg * av + Sv) / degc
        msq = (deg * av * av + 2.0 * av * Sv + q_ref[...]) / degc
        std = jnp.sqrt(jax.nn.relu(msq - mean * mean) + 1e-5)
        has = deg > 0.0
        mn = jnp.where(has, av + mn_ref[...], 0.0)
        mx = jnp.where(has, av + mx_ref[...], 0.0)
        cat = jnp.concatenate([mean, mn, mx, std], axis=1)
        cat1 = jnp.concatenate([s1 * mean, s1 * mn, s1 * mx, s1 * std], axis=1)
        cat2 = jnp.concatenate([s2 * mean, s2 * mn, s2 * mx, s2 * std], axis=1)
        o = jnp.dot(h_ref[...], qx_ref[...], preferred_element_type=jnp.float32)
        o = o + jnp.dot(cat, q1_ref[...], preferred_element_type=jnp.float32)
        o = o + jnp.dot(cat1, q2_ref[...], preferred_element_type=jnp.float32)
        o = o + jnp.dot(cat2, q3_ref[...], preferred_element_type=jnp.float32)
        o = o + qb_ref[0:1, :]
        y_ref[...] = jnp.dot(o, lw_ref[...], preferred_element_type=jnp.float32) + lb_ref[0:1, :]

    full = lambda shape: pl.BlockSpec(shape, lambda i: (0, 0))
    return pl.pallas_call(
        body,
        grid=grid,
        in_specs=[
            pl.BlockSpec((R, 128), lambda i: (i, 0)),
            pl.BlockSpec((R, TF), lambda i: (i, 0)),
            pl.BlockSpec((R, TF), lambda i: (i, 0)),
            pl.BlockSpec((R, TF), lambda i: (i, 0)),
            pl.BlockSpec((R, TF), lambda i: (i, 0)),
            pl.BlockSpec((R, TF), lambda i: (i, 0)),
            pl.BlockSpec((R, TF), lambda i: (i, 0)),
            full((128, 128)),
            full((4 * TF, 128)),
            full((4 * TF, 128)),
            full((4 * TF, 128)),
            full((8, 128)),
            full((128, 128)),
            full((8, 128)),
        ],
        out_specs=pl.BlockSpec((R, 128), lambda i: (i, 0)),
        out_shape=jax.ShapeDtypeStruct((NP, 128), jnp.float32),
    )(h, a, S, Q, MN, MX, degb, Qx, Q1, Q2, Q3, qbf, lWp, lbf)


def _tc_bn(y, bnw, bnb):
    def body(y_ref, w_ref, b_ref, h_ref):
        yv = y_ref[...]
        mask = lax.broadcasted_iota(jnp.int32, (NP, 128), 0) < N
        ym = jnp.where(mask, yv, 0.0)
        mu = jnp.sum(ym, axis=0, keepdims=True) / N
        va = jnp.sum(ym * ym, axis=0, keepdims=True) / N - mu * mu
        hv = jax.nn.relu((yv - mu) * lax.rsqrt(va + 1e-5) * w_ref[0:1, :] + b_ref[0:1, :])
        h_ref[...] = jnp.where(mask, hv, 0.0)

    return pl.pallas_call(
        body,
        out_shape=jax.ShapeDtypeStruct((NP, 128), jnp.float32),
    )(y, bnw, bnb)


def _tc_final(h, oneB, W1p, b1p, W2p, b2p, W3p, b3p):
    def body(h_ref, ob_ref, w1, b1, w2, b2, w3, b3, o_ref):
        pooled = lax.dot_general(ob_ref[...], h_ref[...], (((0,), (0,)), ((), ())),
                                 preferred_element_type=jnp.float32)
        z = jax.nn.relu(jnp.dot(pooled, w1[...], preferred_element_type=jnp.float32) + b1[0:1, :])
        z = jax.nn.relu(jnp.dot(z, w2[...], preferred_element_type=jnp.float32) + b2[0:1, :])
        o_ref[...] = jnp.dot(z, w3[...], preferred_element_type=jnp.float32) + b3[0:1, :]

    return pl.pallas_call(
        body,
        out_shape=jax.ShapeDtypeStruct((128, 128), jnp.float32),
    )(h, oneB, W1p, b1p, W2p, b2p, W3p, b3p)


def _sc_edge_pass(u, vflat, src_s, ad_s, rp):
    mesh = plsc.VectorSubcoreMesh(core_axis_name="c", subcore_axis_name="s")
    info = plsc.get_sparse_core_info()
    NC = info.num_cores

    @functools.partial(
        pl.kernel,
        out_type=[jax.ShapeDtypeStruct((NP, TF), jnp.float32)] * 4,
        mesh=mesh,
        scratch_types=[
            pltpu.VMEM((336,), jnp.int32),
            pltpu.VMEM((4 * TF,), jnp.float32),
            pltpu.VMEM((2, CH), jnp.int32),
            pltpu.VMEM((2, CH), jnp.int32),
            pltpu.VMEM((2, CH, TF), jnp.float32),
            pltpu.VMEM((CELL + 1, TF), jnp.float32),
            pltpu.VMEM((CELL + 1, TF), jnp.float32),
            pltpu.VMEM((CELL + 1, TF), jnp.float32),
            pltpu.VMEM((CELL + 1, TF), jnp.float32),
            pltpu.SemaphoreType.DMA((2,)),
        ],
    )
    def k(u_hbm, v_hbm, src_hbm, ad_hbm, rp_hbm,
          s_hbm, q_hbm, mn_hbm, mx_hbm,
          rpv, vt, idx2, ad2, rows2, accS, accQ, accMN, accMX, sem):
        w = lax.axis_index("s") * NC + lax.axis_index("c")
        nbase_w = w * NODES_PER_W
        pltpu.sync_copy(rp_hbm.at[pl.ds(nbase_w, 336)], rpv)
        pltpu.sync_copy(v_hbm, vt)

        def cell_body(cell, _):
            nb_l = cell * CELL
            abs_base = nbase_w + nb_l
            e_lo = rpv[pl.ds(nb_l, 16)][0]
            e_hi = rpv[pl.ds(nb_l + CELL, 16)][0]

            zero = jnp.zeros((16,), jnp.float32)
            big = jnp.full((16,), BIG, jnp.float32)

            def zbody(r, _):
                for j in range(TF // 16):
                    accS[r, pl.ds(j * 16, 16)] = zero
                    accQ[r, pl.ds(j * 16, 16)] = zero
                    accMN[r, pl.ds(j * 16, 16)] = big
                    accMX[r, pl.ds(j * 16, 16)] = -big
                return 0

            lax.fori_loop(0, CELL + 1, zbody, 0)

            c0 = e_lo // CH
            c1 = (e_hi + CH - 1) // CH
            nch = c1 - c0

            def start_chunk(c, slot):
                pltpu.sync_copy(src_hbm.at[pl.ds(c * CH, CH)], idx2.at[slot])
                pltpu.sync_copy(ad_hbm.at[pl.ds(c * CH, CH)], ad2.at[slot])
                pltpu.async_copy(u_hbm.at[idx2.at[slot]], rows2.at[slot], sem.at[slot])

            @pl.when(nch > 0)
            def _():
                start_chunk(c0, 0)

            def chunk_body(c, _):
                par = lax.rem(c - c0, 2)
                alt = 1 - par

                @pl.when(c + 1 < c1)
                def _():
                    start_chunk(c + 1, alt)

                pltpu.make_async_copy(
                    u_hbm.at[idx2.at[par]], rows2.at[par], sem.at[par]).wait()

                adv = ad2[par]
                base = c * CH
                dls = []
                vofs = []
                for i in range(CH):
                    adi = adv[i]
                    d_loc = (adi >> 2) - abs_base
                    valid = jnp.logical_and(base + i >= e_lo, base + i < e_hi)
                    dls.append(jnp.where(valid, d_loc, CELL))
                    vofs.append((adi & 3) * TF)

                def jbody(j, _):
                    col = j * 16
                    for i in range(CH):
                        uv = rows2[par, i, pl.ds(col, 16)]
                        vv = vt[pl.ds(vofs[i] + col, 16)]
                        g = uv + vv
                        plsc.addupdate(accS.at[dls[i], pl.ds(col, 16)], g)
                        plsc.addupdate(accQ.at[dls[i], pl.ds(col, 16)], g * g)
                        m0 = accMN[dls[i], pl.ds(col, 16)]
                        accMN[dls[i], pl.ds(col, 16)] = jnp.minimum(m0, g)
                        m1 = accMX[dls[i], pl.ds(col, 16)]
                        accMX[dls[i], pl.ds(col, 16)] = jnp.maximum(m1, g)
                    return 0

                lax.fori_loop(0, TF // 16, jbody, 0)
                return 0

            lax.fori_loop(c0, c1, chunk_body, 0)

            pltpu.sync_copy(accS.at[pl.ds(0, CELL)], s_hbm.at[pl.ds(abs_base, CELL)])
            pltpu.sync_copy(accQ.at[pl.ds(0, CELL)], q_hbm.at[pl.ds(abs_base, CELL)])
            pltpu.sync_copy(accMN.at[pl.ds(0, CELL)], mn_hbm.at[pl.ds(abs_base, CELL)])
            pltpu.sync_copy(accMX.at[pl.ds(0, CELL)], mx_hbm.at[pl.ds(abs_base, CELL)])
            return 0

        lax.fori_loop(0, NODES_PER_W // CELL, cell_body, 0)

    return k(u, vflat, src_s, ad_s, rp)


def _pad2(m, r, c):
    return jnp.zeros((r, c), jnp.float32).at[:m.shape[0], :m.shape[1]].set(m)


def _row8(v, c):
    return jnp.zeros((8, c), jnp.float32).at[0, :v.shape[0]].set(v)


def kernel(x, edge_index, edge_attr, batch, node_emb, edge_emb, edge_enc_W, edge_enc_b, pre_W, pre_b, post_W, post_b, lin_W, lin_b, bn_w, bn_b, mlp_W1, mlp_b1, mlp_W2, mlp_b2, mlp_W3, mlp_b3):
    src, dst = edge_index[0], edge_index[1]
    perm = jnp.argsort(dst)
    src_s = src[perm].astype(jnp.int32)
    dst_s = dst[perm].astype(jnp.int32)
    attr_s = edge_attr[perm].astype(jnp.int32)
    ad_s = dst_s * 4 + attr_s
    rp = jnp.searchsorted(dst_s, jnp.arange(NP + 1)).astype(jnp.int32)
    rp_pad = jnp.concatenate([rp, jnp.full((15,), E, jnp.int32)])
    deg = (rp[1:] - rp[:-1]).astype(jnp.float32)
    degb = jnp.broadcast_to(deg[:, None], (NP, TF))

    xo = jnp.zeros((NP, 128), jnp.float32).at[:N].set(
        (x[:, None] == jnp.arange(128)[None, :]).astype(jnp.float32))
    oneB = jnp.zeros((NP, 128), jnp.float32).at[:N].set(
        (batch[:, None] == jnp.arange(128)[None, :]).astype(jnp.float32))
    emb_pad = _pad2(node_emb, 128, F)

    h = None
    for l in range(L):
        pW, pb, qW, qb = pre_W[l], pre_b[l], post_W[l], post_b[l]
        Wd, Ws, We = pW[:, :F, :], pW[:, F:2 * F, :], pW[:, 2 * F:, :]
        Wsf75 = Ws.transpose(1, 0, 2).reshape(F, T * F)
        Wdf75 = Wd.transpose(1, 0, 2).reshape(F, T * F)
        Qx75 = qW[:, :F, :].transpose(1, 0, 2).reshape(F, T * 15)
        if l == 0:
            Wsf = _pad2(emb_pad @ Wsf75, 128, TF)
            Wdf = _pad2(emb_pad @ Wdf75, 128, TF)
            Qx = _pad2(emb_pad @ Qx75, 128, 128)
            hmat = xo
        else:
            Wsf = _pad2(Wsf75, 128, TF)
            Wdf = _pad2(Wdf75, 128, TF)
            Qx = _pad2(Qx75, 128, 128)
            hmat = h
        pbf = _row8(pb.reshape(T * F), TF)
        v4 = (edge_emb @ edge_enc_W[l] + edge_enc_b[l]) @ We.transpose(1, 0, 2).reshape(F, T * F)
        vflat = _pad2(v4, 4, TF).reshape(4 * TF)

        Qs = []
        for grp in range(3):
            Qm = jnp.zeros((4 * TF, 128), jnp.float32)
            for s in range(4):
                for t in range(T):
                    blkw = qW[t, F * (1 + 4 * grp + s):F * (2 + 4 * grp + s), :]
                    Qm = Qm.at[s * TF + t * F:s * TF + (t + 1) * F,
                               t * 15:(t + 1) * 15].set(blkw)
            Qs.append(Qm)
        qbf = _row8(qb.reshape(T * 15), 128)
        lWp = _pad2(lin_W[l], 128, 128)
        lbf = _row8(lin_b[l], 128)

        u, a = _tc_pre(hmat, Wsf, Wdf, pbf)
        S, Q, MN, MX = _sc_edge_pass(u, vflat, src_s, ad_s, rp_pad)
        y = _tc_post(hmat, a, S, Q, MN, MX, degb, Qx, Qs[0], Qs[1], Qs[2], qbf, lWp, lbf)
        h = _tc_bn(y, _row8(bn_w[l], 128), _row8(bn_b[l], 128))

    out = _tc_final(h, oneB, _pad2(mlp_W1, 128, 128), _row8(mlp_b1, 128),
                    _pad2(mlp_W2, 128, 128), _row8(mlp_b2, 128),
                    _pad2(mlp_W3, 128, 128), _row8(mlp_b3, 128))
    return out[:, 0:1]
